# Initial kernel scaffold; baseline (speedup 1.0000x reference)
import jax
import jax.numpy as jnp
from jax.experimental import pallas as pl


def kernel(query, key, value, proj_dir, sampled_set):
    raise NotImplementedError("write your pallas kernel here")


# speedup vs baseline: 3.4067x; 3.4067x over previous
import functools

import jax
import jax.numpy as jnp
import numpy as np
from jax.experimental import pallas as pl
from jax.experimental.pallas import tpu as pltpu

LSH_NUM_PROJS = 8
BLOCK_SIZE = 256
SAMPLE_SIZE = 256


def _unit_hamming_distance_array(size_n):
    a = np.array([0, 1], dtype=np.int32)
    for _ in range(size_n - 1):
        a = np.concatenate([a, a[::-1] + a.shape[0]], axis=0)
    return a


_PERM_NP = _unit_hamming_distance_array(LSH_NUM_PROJS)


def _attn_body(q_ref, k_ref, v_ref, ks_ref, vs_ref, samp_ref, o_ref, *, scale, n_total):
    j = pl.program_id(1)
    q = q_ref[0]
    k = k_ref[0]
    v = v_ref[0]
    ks = ks_ref[0]
    vs = vs_ref[0]
    samp = samp_ref[0, 0]

    s1 = jax.lax.dot_general(q, k, (((1,), (1,)), ((), ())),
                             preferred_element_type=jnp.float32) * scale
    m1 = jnp.max(s1, axis=-1, keepdims=True)
    e1 = jnp.exp(s1 - m1)
    d1 = jnp.sum(e1, axis=-1, keepdims=True)
    lse1 = m1 + jnp.log(d1)
    a1 = jnp.dot(e1 / d1, v, preferred_element_type=jnp.float32)

    s2 = jax.lax.dot_general(q, ks, (((1,), (1,)), ((), ())),
                             preferred_element_type=jnp.float32) * scale
    bias = jnp.where((samp // BLOCK_SIZE) == j,
                     jnp.finfo(jnp.float32).min, 0.0)
    s2 = s2 + bias[None, :]
    m2 = jnp.max(s2, axis=-1, keepdims=True)
    e2 = jnp.exp(s2 - m2)
    d2 = jnp.sum(e2, axis=-1, keepdims=True)
    lse2 = m2 + jnp.log(d2) + jnp.log(n_total / SAMPLE_SIZE)
    a2 = jnp.dot(e2 / d2, vs, preferred_element_type=jnp.float32)

    c = 1.0 / (1.0 + jnp.exp(lse2 - lse1))
    o_ref[0] = c * a1 + (1.0 - c) * a2


def _block_attention(q_sorted, k_sorted, v_sorted, k_subset, v_subset, sampled_set):
    G, N, D = q_sorted.shape
    nb = N // BLOCK_SIZE
    grid = (G, nb)
    body = functools.partial(_attn_body, scale=D ** (-0.5), n_total=float(N))
    return pl.pallas_call(
        body,
        grid=grid,
        in_specs=[
            pl.BlockSpec((1, BLOCK_SIZE, D), lambda i, j: (i, j, 0)),
            pl.BlockSpec((1, BLOCK_SIZE, D), lambda i, j: (i, j, 0)),
            pl.BlockSpec((1, BLOCK_SIZE, D), lambda i, j: (i, j, 0)),
            pl.BlockSpec((1, SAMPLE_SIZE, D), lambda i, j: (i, 0, 0)),
            pl.BlockSpec((1, SAMPLE_SIZE, D), lambda i, j: (i, 0, 0)),
            pl.BlockSpec((1, 1, SAMPLE_SIZE), lambda i, j: (i, 0, 0)),
        ],
        out_specs=pl.BlockSpec((1, BLOCK_SIZE, D), lambda i, j: (i, j, 0)),
        out_shape=jax.ShapeDtypeStruct((G, N, D), jnp.float32),
    )(q_sorted, k_sorted, v_sorted, k_subset, v_subset, sampled_set)


def kernel(query, key, value, proj_dir, sampled_set):
    B, N, H, D = query.shape
    G = B * H
    perm = jnp.asarray(_PERM_NP)
    mask = (2 ** jnp.arange(LSH_NUM_PROJS)).astype(jnp.int32)

    qt = jnp.transpose(query, (0, 2, 1, 3)).reshape(G, N, D)
    kt = jnp.transpose(key, (0, 2, 1, 3)).reshape(G, N, D)
    vt = jnp.transpose(value, (0, 2, 1, 3)).reshape(G, N, D)

    def lsh(x):
        proj = jnp.einsum('gnd,dp->gnp', x, proj_dir)
        bin_ids = jnp.sum((proj > 0).astype(jnp.int32) * mask, axis=-1)
        return perm[bin_ids]

    q_hash = lsh(qt)
    k_hash = lsh(kt)
    q_sort_idx = jnp.argsort(q_hash, axis=1, stable=True)
    k_sort_idx = jnp.argsort(k_hash, axis=1, stable=True)

    q_sorted = jnp.take_along_axis(qt, q_sort_idx[..., None], axis=1)
    k_sorted = jnp.take_along_axis(kt, k_sort_idx[..., None], axis=1)
    v_sorted = jnp.take_along_axis(vt, k_sort_idx[..., None], axis=1)

    samp = sampled_set.reshape(G, SAMPLE_SIZE)
    sub_idx = jnp.take_along_axis(k_sort_idx, samp, axis=1)
    k_subset = jnp.take_along_axis(kt, sub_idx[..., None], axis=1)
    v_subset = jnp.take_along_axis(vt, sub_idx[..., None], axis=1)

    attn_sorted = _block_attention(q_sorted, k_sorted, v_sorted,
                                   k_subset, v_subset,
                                   samp.reshape(G, 1, SAMPLE_SIZE))

    q_sort_idx_inv = jnp.argsort(q_sort_idx, axis=1, stable=True)
    attn = jnp.take_along_axis(attn_sorted, q_sort_idx_inv[..., None], axis=1)
    return jnp.transpose(attn.reshape(B, H, N, D), (0, 2, 1, 3))
